# baseline (device time: 12084 ns/iter reference)
import jax
import jax.numpy as jnp
from jax import lax
from jax.experimental import pallas as pl
from jax.experimental.pallas import tpu as pltpu

N_DEV = 4
B, Sq, SKV_LOC, Hq, Dh = 2, 128, 128, 4, 64
D_MODEL = 512
D_QK = Hq * Dh

EVENS = (0, 2)
SEND_TARGETS = {0: (1, 2, 3), 2: (3, 0, 1)}


def kernel(x, Wq, K_ext, V_ext, Wo):
    Q = jnp.dot(x.reshape(B * Sq, D_MODEL), Wq,
                preferred_element_type=jnp.float32)
    KT = jnp.transpose(K_ext, (0, 2, 3, 1))
    VT = jnp.transpose(V_ext, (0, 2, 3, 1))

    def body(q_ref, kt_ref, vt_ref, ctx_ref,
             o_all, ms_all, send_sems, recv_sems):
        my = lax.axis_index("i")

        barrier = pltpu.get_barrier_semaphore()
        for tgt in EVENS:
            @pl.when(my != tgt)
            def _():
                pl.semaphore_signal(barrier, inc=1, device_id=(tgt,),
                                    device_id_type=pl.DeviceIdType.MESH)

        def compute_partial(slot, dev):
            for b in range(B):
                for h in range(Hq):
                    qbh = q_ref[b * Sq:(b + 1) * Sq, h * Dh:(h + 1) * Dh]
                    s = jnp.dot(qbh, kt_ref[b, h],
                                preferred_element_type=jnp.float32) * 0.125
                    parts = []
                    for blk in range(2):
                        sb = s[blk * 64:(blk + 1) * 64,
                               blk * 64:(blk + 1) * 64]
                        m = jnp.max(sb, axis=1)
                        w = jnp.exp(sb - m[:, None])
                        ssum = jnp.sum(w, axis=1)
                        o = lax.dot_general(
                            w, vt_ref[b, h][:, blk * 64:(blk + 1) * 64],
                            (((1,), (1,)), ((), ())),
                            preferred_element_type=jnp.float32)
                        o_all[slot, b, blk * 64:(blk + 1) * 64,
                              h * Dh:(h + 1) * Dh] = o.astype(jnp.bfloat16)
                        parts.append((m, ssum))
                    ms_all[slot, b, h, 0] = jnp.concatenate(
                        (parts[0][0], parts[1][0]))
                    ms_all[slot, b, h, 1] = jnp.concatenate(
                        (parts[0][1], parts[1][1]))

        def make_rdmas(slot, dev):
            rdmas = []
            for j, tgt in enumerate(SEND_TARGETS[dev]):
                for t, buf in enumerate((o_all, ms_all)):
                    rdmas.append(pltpu.make_async_remote_copy(
                        src_ref=buf.at[slot],
                        dst_ref=buf.at[slot],
                        send_sem=send_sems.at[j, t],
                        recv_sem=recv_sems.at[slot, t],
                        device_id=(tgt,),
                        device_id_type=pl.DeviceIdType.MESH,
                    ))
            return rdmas

        for slot, dev in enumerate(EVENS):
            @pl.when(my == dev)
            def _(slot=slot, dev=dev):
                pl.semaphore_wait(barrier, 3)
                compute_partial(slot, dev)
                for rdma in make_rdmas(slot, dev):
                    rdma.start()

        for slot, dev in enumerate(EVENS):
            @pl.when(my != dev)
            def _(slot=slot):
                for t, buf in enumerate((o_all, ms_all)):
                    recv = pltpu.make_async_remote_copy(
                        src_ref=buf.at[slot],
                        dst_ref=buf.at[slot],
                        send_sem=send_sems.at[0, t],
                        recv_sem=recv_sems.at[slot, t],
                        device_id=(0,),
                        device_id_type=pl.DeviceIdType.MESH,
                    )
                    recv.wait_recv()

        m0 = ms_all[0, :, :, 0]
        m1 = ms_all[1, :, :, 0]
        s0 = ms_all[0, :, :, 1]
        s1 = ms_all[1, :, :, 1]
        m_g = jnp.maximum(m0, m1)
        e0 = jnp.exp(m0 - m_g)
        e1 = jnp.exp(m1 - m_g)
        denom = s0 * e0 + s1 * e1
        c0 = e0 / denom
        c1 = e1 / denom

        def expand(c):
            return jnp.concatenate(
                [jnp.broadcast_to(c[:, h, :, None], (B, Sq, Dh))
                 for h in range(Hq)], axis=-1)

        ctx = (o_all[0].astype(jnp.float32) * expand(c0)
               + o_all[1].astype(jnp.float32) * expand(c1))
        ctx_ref[...] = ctx.reshape(B * Sq, D_QK)

        for slot, dev in enumerate(EVENS):
            @pl.when(my == dev)
            def _(slot=slot, dev=dev):
                for rdma in make_rdmas(slot, dev):
                    rdma.wait_send()

    ctx = pl.pallas_call(
        body,
        out_shape=jax.ShapeDtypeStruct((B * Sq, D_QK), jnp.float32),
        in_specs=[pl.BlockSpec(memory_space=pltpu.MemorySpace.VMEM)] * 3,
        out_specs=pl.BlockSpec(memory_space=pltpu.MemorySpace.VMEM),
        scratch_shapes=[
            pltpu.VMEM((2, B, Sq, D_QK), jnp.bfloat16),
            pltpu.VMEM((2, B, Hq, 2, Sq), jnp.float32),
            pltpu.SemaphoreType.DMA((3, 2)),
            pltpu.SemaphoreType.DMA((2, 2)),
        ],
        compiler_params=pltpu.CompilerParams(collective_id=0),
    )(Q, KT, VT)

    out = jnp.dot(ctx, Wo, preferred_element_type=jnp.float32)
    return out.reshape(B, Sq, D_MODEL)


# device time: 10701 ns/iter; 1.1292x vs baseline; 1.1292x over previous
import jax
import jax.numpy as jnp
from jax import lax
from jax.experimental import pallas as pl
from jax.experimental.pallas import tpu as pltpu

N_DEV = 4
B, Sq, SKV_LOC, Hq, Dh = 2, 128, 128, 4, 64
D_MODEL = 512
D_QK = Hq * Dh

EVENS = (0, 2)
SEND_TARGETS = {0: (1, 2, 3), 2: (3, 0, 1)}


def kernel(x, Wq, K_ext, V_ext, Wo):
    Q = jnp.dot(x.reshape(B * Sq, D_MODEL), Wq,
                preferred_element_type=jnp.float32)
    KT = jnp.transpose(K_ext, (0, 2, 3, 1))
    VT = jnp.transpose(V_ext, (0, 2, 3, 1))

    def body(q_ref, kt_ref, vt_ref, ctx_ref,
             o_all, ms_all, send_sems, recv_sems):
        my = lax.axis_index("i")

        barrier = pltpu.get_barrier_semaphore()
        for tgt in EVENS:
            @pl.when(my != tgt)
            def _():
                pl.semaphore_signal(barrier, inc=1, device_id=(tgt,),
                                    device_id_type=pl.DeviceIdType.MESH)

        def compute_partial_b(slot, dev, b):
            qb = lax.broadcasted_iota(jnp.int32, (Sq, SKV_LOC), 0) // 64
            kb = lax.broadcasted_iota(jnp.int32, (Sq, SKV_LOC), 1) // 64 + 2 * dev
            mask = (qb == kb) | ((kb % 4) == (qb % 4))
            for h in range(Hq):
                qbh = q_ref[b * Sq:(b + 1) * Sq, h * Dh:(h + 1) * Dh]
                s = jnp.dot(qbh, kt_ref[b, h],
                            preferred_element_type=jnp.float32) * 0.125
                s = jnp.where(mask, s, -1e9)
                m = jnp.max(s, axis=1)
                w = jnp.exp(s - m[:, None])
                ssum = jnp.sum(w, axis=1)
                o = lax.dot_general(
                    w, vt_ref[b, h], (((1,), (1,)), ((), ())),
                    preferred_element_type=jnp.float32)
                o_all[slot, b, :, h * Dh:(h + 1) * Dh] = o.astype(jnp.bfloat16)
                ms_all[slot, b, h, 0] = m
                ms_all[slot, b, h, 1] = ssum

        def make_rdmas(slot, dev, b):
            rdmas = []
            for j, tgt in enumerate(SEND_TARGETS[dev]):
                for t, buf in enumerate((o_all, ms_all)):
                    rdmas.append(pltpu.make_async_remote_copy(
                        src_ref=buf.at[slot, b],
                        dst_ref=buf.at[slot, b],
                        send_sem=send_sems.at[j, t, b],
                        recv_sem=recv_sems.at[slot, t, b],
                        device_id=(tgt,),
                        device_id_type=pl.DeviceIdType.MESH,
                    ))
            return rdmas

        for slot, dev in enumerate(EVENS):
            @pl.when(my == dev)
            def _(slot=slot, dev=dev):
                pl.semaphore_wait(barrier, 3)
                for b in range(B):
                    compute_partial_b(slot, dev, b)
                    for rdma in make_rdmas(slot, dev, b):
                        rdma.start()

        for slot, dev in enumerate(EVENS):
            @pl.when(my != dev)
            def _(slot=slot):
                for b in range(B):
                    for t, buf in enumerate((o_all, ms_all)):
                        recv = pltpu.make_async_remote_copy(
                            src_ref=buf.at[slot, b],
                            dst_ref=buf.at[slot, b],
                            send_sem=send_sems.at[0, t, b],
                            recv_sem=recv_sems.at[slot, t, b],
                            device_id=(0,),
                            device_id_type=pl.DeviceIdType.MESH,
                        )
                        recv.wait_recv()

        m0 = ms_all[0, :, :, 0]
        m1 = ms_all[1, :, :, 0]
        s0 = ms_all[0, :, :, 1]
        s1 = ms_all[1, :, :, 1]
        m_g = jnp.maximum(m0, m1)
        e0 = jnp.exp(m0 - m_g)
        e1 = jnp.exp(m1 - m_g)
        denom = s0 * e0 + s1 * e1
        c0 = e0 / denom
        c1 = e1 / denom

        def expand(c):
            return jnp.concatenate(
                [jnp.broadcast_to(c[:, h, :, None], (B, Sq, Dh))
                 for h in range(Hq)], axis=-1)

        ctx = (o_all[0].astype(jnp.float32) * expand(c0)
               + o_all[1].astype(jnp.float32) * expand(c1))
        ctx_ref[...] = ctx.reshape(B * Sq, D_QK)

        for slot, dev in enumerate(EVENS):
            @pl.when(my == dev)
            def _(slot=slot, dev=dev):
                for b in range(B):
                    for rdma in make_rdmas(slot, dev, b):
                        rdma.wait_send()

    ctx = pl.pallas_call(
        body,
        out_shape=jax.ShapeDtypeStruct((B * Sq, D_QK), jnp.float32),
        in_specs=[pl.BlockSpec(memory_space=pltpu.MemorySpace.VMEM)] * 3,
        out_specs=pl.BlockSpec(memory_space=pltpu.MemorySpace.VMEM),
        scratch_shapes=[
            pltpu.VMEM((2, B, Sq, D_QK), jnp.bfloat16),
            pltpu.VMEM((2, B, Hq, 2, Sq), jnp.float32),
            pltpu.SemaphoreType.DMA((3, 2, B)),
            pltpu.SemaphoreType.DMA((2, 2, B)),
        ],
        compiler_params=pltpu.CompilerParams(collective_id=0),
    )(Q, KT, VT)

    out = jnp.dot(ctx, Wo, preferred_element_type=jnp.float32)
    return out.reshape(B, Sq, D_MODEL)


# device time: 10407 ns/iter; 1.1611x vs baseline; 1.0283x over previous
import jax
import jax.numpy as jnp
from jax import lax
from jax.experimental import pallas as pl
from jax.experimental.pallas import tpu as pltpu

N_DEV = 4
B, Sq, SKV_LOC, Hq, Dh = 2, 128, 128, 4, 64
D_MODEL = 512
D_QK = Hq * Dh

EVENS = (0, 2)
SEND_TARGETS = {0: (1, 2, 3), 2: (3, 0, 1)}


def kernel(x, Wq, K_ext, V_ext, Wo):
    Q = jnp.dot(x.reshape(B * Sq, D_MODEL), Wq,
                preferred_element_type=jnp.float32)
    KT = jnp.transpose(K_ext, (0, 2, 3, 1))
    VT = jnp.transpose(V_ext, (0, 2, 3, 1))

    def body(q_ref, kt_ref, vt_ref, ctx_ref,
             o_all, ms_all, send_sems, recv_sems):
        my = lax.axis_index("i")

        barrier = pltpu.get_barrier_semaphore()
        for tgt in EVENS:
            @pl.when(my != tgt)
            def _():
                pl.semaphore_signal(barrier, inc=1, device_id=(tgt,),
                                    device_id_type=pl.DeviceIdType.MESH)

        def compute_partial(slot, dev):
            qb = lax.broadcasted_iota(jnp.int32, (Sq, SKV_LOC), 0) // 64
            kb = lax.broadcasted_iota(jnp.int32, (Sq, SKV_LOC), 1) // 64 + 2 * dev
            mask = (qb == kb) | ((kb % 4) == (qb % 4))
            for b in range(B):
                for h in range(Hq):
                    qbh = q_ref[b * Sq:(b + 1) * Sq, h * Dh:(h + 1) * Dh]
                    s = jnp.dot(qbh, kt_ref[b, h],
                                preferred_element_type=jnp.float32) * 0.125
                    s = jnp.where(mask, s, -1e9)
                    m = jnp.max(s, axis=1)
                    w = jnp.exp(s - m[:, None])
                    ssum = jnp.sum(w, axis=1)
                    o = lax.dot_general(
                        w, vt_ref[b, h], (((1,), (1,)), ((), ())),
                        preferred_element_type=jnp.float32)
                    o_all[slot, b, :, h * Dh:(h + 1) * Dh] = o.astype(
                        jnp.bfloat16)
                    ms_all[slot, b, h, 0] = m
                    ms_all[slot, b, h, 1] = ssum

        def make_rdmas(slot, dev):
            rdmas = []
            for j, tgt in enumerate(SEND_TARGETS[dev]):
                for t, buf in enumerate((o_all, ms_all)):
                    rdmas.append(pltpu.make_async_remote_copy(
                        src_ref=buf.at[slot],
                        dst_ref=buf.at[slot],
                        send_sem=send_sems.at[j, t],
                        recv_sem=recv_sems.at[slot, t],
                        device_id=(tgt,),
                        device_id_type=pl.DeviceIdType.MESH,
                    ))
            return rdmas

        for slot, dev in enumerate(EVENS):
            @pl.when(my == dev)
            def _(slot=slot, dev=dev):
                pl.semaphore_wait(barrier, 3)
                compute_partial(slot, dev)
                for rdma in make_rdmas(slot, dev):
                    rdma.start()

        for slot, dev in enumerate(EVENS):
            @pl.when(my != dev)
            def _(slot=slot):
                for t, buf in enumerate((o_all, ms_all)):
                    recv = pltpu.make_async_remote_copy(
                        src_ref=buf.at[slot],
                        dst_ref=buf.at[slot],
                        send_sem=send_sems.at[0, t],
                        recv_sem=recv_sems.at[slot, t],
                        device_id=(0,),
                        device_id_type=pl.DeviceIdType.MESH,
                    )
                    recv.wait_recv()

        m0 = ms_all[0, :, :, 0]
        m1 = ms_all[1, :, :, 0]
        s0 = ms_all[0, :, :, 1]
        s1 = ms_all[1, :, :, 1]
        m_g = jnp.maximum(m0, m1)
        e0 = jnp.exp(m0 - m_g)
        e1 = jnp.exp(m1 - m_g)
        denom = s0 * e0 + s1 * e1
        c0 = e0 / denom
        c1 = e1 / denom

        def expand(c):
            return jnp.concatenate(
                [jnp.broadcast_to(c[:, h, :, None], (B, Sq, Dh))
                 for h in range(Hq)], axis=-1)

        ctx = (o_all[0].astype(jnp.float32) * expand(c0)
               + o_all[1].astype(jnp.float32) * expand(c1))
        ctx_ref[...] = ctx.reshape(B * Sq, D_QK).astype(jnp.bfloat16)

        for slot, dev in enumerate(EVENS):
            @pl.when(my == dev)
            def _(slot=slot, dev=dev):
                for rdma in make_rdmas(slot, dev):
                    rdma.wait_send()

    ctx = pl.pallas_call(
        body,
        out_shape=jax.ShapeDtypeStruct((B * Sq, D_QK), jnp.bfloat16),
        in_specs=[pl.BlockSpec(memory_space=pltpu.MemorySpace.VMEM)] * 3,
        out_specs=pl.BlockSpec(memory_space=pltpu.MemorySpace.VMEM),
        scratch_shapes=[
            pltpu.VMEM((2, B, Sq, D_QK), jnp.bfloat16),
            pltpu.VMEM((2, B, Hq, 2, Sq), jnp.float32),
            pltpu.SemaphoreType.DMA((3, 2)),
            pltpu.SemaphoreType.DMA((2, 2)),
        ],
        compiler_params=pltpu.CompilerParams(collective_id=0),
    )(Q, KT, VT)

    out = jnp.dot(ctx, Wo, preferred_element_type=jnp.float32)
    return out.reshape(B, Sq, D_MODEL)
